# baseline (speedup 1.0000x reference)
import functools

import jax
import jax.numpy as jnp
from jax import lax
from jax.experimental import pallas as pl
from jax.experimental.pallas import tpu as pltpu
from jax.experimental.pallas import tpu_sc as plsc

_NUM_CORES = 2
_NUM_SUBCORES = 16
_NUM_WORKERS = _NUM_CORES * _NUM_SUBCORES


def kernel(x, emb):
    seq_len = x.shape[1]
    emb_dim = emb.shape[1]
    rows_per_worker = seq_len // _NUM_WORKERS

    mesh = plsc.VectorSubcoreMesh(core_axis_name="c", subcore_axis_name="s")

    chunk = 8
    n_chunks = rows_per_worker // chunk
    nbuf = 14

    @functools.partial(
        pl.kernel,
        mesh=mesh,
        out_type=jax.ShapeDtypeStruct((seq_len, emb_dim), emb.dtype),
        scratch_types=[
            pltpu.VMEM((nbuf, chunk, emb_dim), jnp.float32),
            pltpu.SemaphoreType.DMA,
            pltpu.SemaphoreType.DMA,
        ],
    )
    def sc_copy(emb_hbm, out_hbm, buf, sem_in, sem_out):
        wid = lax.axis_index("s") * _NUM_CORES + lax.axis_index("c")
        base = wid * rows_per_worker

        def in_cp(i, b):
            return pltpu.make_async_copy(
                emb_hbm.at[pl.ds(base + i * chunk, chunk)], buf.at[b], sem_in
            )

        def out_cp(i, b):
            return pltpu.make_async_copy(
                buf.at[b], out_hbm.at[pl.ds(base + i * chunk, chunk)], sem_out
            )

        for j in range(min(nbuf, n_chunks)):
            in_cp(j, j).start()
        for i in range(n_chunks):
            b = i % nbuf
            in_cp(i, b).wait()
            out_cp(i, b).start()
            j = i + nbuf
            if j < n_chunks:
                out_cp(j - nbuf, b).wait()
                in_cp(j, b).start()
        for i in range(max(0, n_chunks - nbuf), n_chunks):
            out_cp(i, i % nbuf).wait()

    return sc_copy(emb)[None]


# speedup vs baseline: 1.0374x; 1.0374x over previous
import functools

import jax
import jax.numpy as jnp
from jax import lax
from jax.experimental import pallas as pl
from jax.experimental.pallas import tpu as pltpu
from jax.experimental.pallas import tpu_sc as plsc

_NUM_CORES = 2
_NUM_SUBCORES = 16
_NUM_WORKERS = _NUM_CORES * _NUM_SUBCORES


def kernel(x, emb):
    seq_len = x.shape[1]
    emb_dim = emb.shape[1]
    rows_per_worker = seq_len // _NUM_WORKERS

    mesh = plsc.VectorSubcoreMesh(core_axis_name="c", subcore_axis_name="s")

    sizes = (8, 40, 40, 40)
    offs = (0, 8, 48, 88)
    bufs = (0, 1, 2, 0)
    buf_rows = max(sizes)
    assert sum(sizes) == rows_per_worker

    @functools.partial(
        pl.kernel,
        mesh=mesh,
        out_type=jax.ShapeDtypeStruct((seq_len, emb_dim), emb.dtype),
        scratch_types=[
            pltpu.VMEM((3, buf_rows, emb_dim), jnp.float32),
            pltpu.SemaphoreType.DMA,
            pltpu.SemaphoreType.DMA,
        ],
    )
    def sc_copy(emb_hbm, out_hbm, buf, sem_in, sem_out):
        wid = lax.axis_index("s") * _NUM_CORES + lax.axis_index("c")
        base = wid * rows_per_worker

        def in_cp(i):
            return pltpu.make_async_copy(
                emb_hbm.at[pl.ds(base + offs[i], sizes[i])],
                buf.at[bufs[i], pl.ds(0, sizes[i])],
                sem_in,
            )

        def out_cp(i):
            return pltpu.make_async_copy(
                buf.at[bufs[i], pl.ds(0, sizes[i])],
                out_hbm.at[pl.ds(base + offs[i], sizes[i])],
                sem_out,
            )

        in_cp(0).start()
        in_cp(1).start()
        in_cp(2).start()
        in_cp(0).wait()
        out_cp(0).start()
        out_cp(0).wait()
        in_cp(3).start()
        for i in (1, 2, 3):
            in_cp(i).wait()
            out_cp(i).start()
        for i in (1, 2, 3):
            out_cp(i).wait()

    return sc_copy(emb)[None]
